# Initial kernel scaffold; baseline (speedup 1.0000x reference)
import jax
import jax.numpy as jnp
from jax.experimental import pallas as pl


def kernel(output, label):
    raise NotImplementedError("write your pallas kernel here")


# speedup vs baseline: 4.5374x; 4.5374x over previous
import jax
import jax.numpy as jnp
from jax import lax
from jax.experimental import pallas as pl
from jax.experimental.pallas import tpu as pltpu

K = 5
ROWS_PER_BLOCK = 8
NUM_ROWS = 128
NUM_COLS = 32768
NUM_BLOCKS = NUM_ROWS // ROWS_PER_BLOCK


def _acc_body(x_ref, lab_ref, out_ref):
    i = pl.program_id(0)
    x = x_ref[...]
    lab = lab_ref[0, 0, :]
    col = lax.broadcasted_iota(jnp.int32, (ROWS_PER_BLOCK, NUM_COLS), 1)
    is_lab = col == lab[:, None]
    v = jnp.max(jnp.where(is_lab, x, -jnp.inf), axis=1)
    ahead = (x > v[:, None]) | ((x == v[:, None]) & (col < lab[:, None]))
    rank = jnp.sum(ahead.astype(jnp.int32), axis=1)
    correct = (rank < K).astype(jnp.float32)

    partial = (jnp.sum(correct) * (1.0 / NUM_ROWS)).reshape(1, 1)

    @pl.when(i == 0)
    def _():
        out_ref[...] = jnp.zeros((1, 1), jnp.float32)

    out_ref[...] += partial


def kernel(output, label):
    lab3 = label.reshape(NUM_BLOCKS, 1, ROWS_PER_BLOCK)
    acc = pl.pallas_call(
        _acc_body,
        grid=(NUM_BLOCKS,),
        in_specs=[
            pl.BlockSpec((ROWS_PER_BLOCK, NUM_COLS), lambda i: (i, 0)),
            pl.BlockSpec((1, 1, ROWS_PER_BLOCK), lambda i: (i, 0, 0)),
        ],
        out_specs=pl.BlockSpec((1, 1), lambda i: (0, 0)),
        out_shape=jax.ShapeDtypeStruct((1, 1), jnp.float32),
    )(output, lab3)
    return acc[0, 0]


# speedup vs baseline: 6.3846x; 1.4071x over previous
import jax
import jax.numpy as jnp
from jax import lax
from jax.experimental import pallas as pl
from jax.experimental.pallas import tpu as pltpu

K = 5
ROWS_PER_BLOCK = 16
NUM_ROWS = 128
NUM_COLS = 32768
NUM_BLOCKS = NUM_ROWS // ROWS_PER_BLOCK
SCALE = 1.0 / NUM_ROWS


def _acc_body(x_ref, lab_ref, out_ref):
    i = pl.program_id(0)
    x = x_ref[...]
    lab = lab_ref[0, 0, :]
    col = lax.broadcasted_iota(jnp.int32, (ROWS_PER_BLOCK, NUM_COLS), 1)
    is_lab = col == lab[:, None]
    v = jnp.max(jnp.where(is_lab, x, -jnp.inf), axis=1)
    vb = v[:, None]
    t = jnp.where(x > vb, 65536, jnp.where(x == vb, 1, 0))
    packed = jnp.sum(t, axis=1)
    gt = packed >> 16
    eq = packed & 0xFFFF

    @pl.when(i == 0)
    def _():
        out_ref[...] = jnp.zeros((1, 1), jnp.float32)

    no_ties = jnp.all(eq == 1)

    @pl.when(no_ties)
    def _():
        correct = (gt < K).astype(jnp.float32)
        out_ref[...] += (jnp.sum(correct) * SCALE).reshape(1, 1)

    @pl.when(jnp.logical_not(no_ties))
    def _():
        eq_before = jnp.sum(
            ((x == vb) & (col < lab[:, None])).astype(jnp.int32), axis=1)
        correct = ((gt + eq_before) < K).astype(jnp.float32)
        out_ref[...] += (jnp.sum(correct) * SCALE).reshape(1, 1)


def kernel(output, label):
    lab3 = label.reshape(NUM_BLOCKS, 1, ROWS_PER_BLOCK)
    acc = pl.pallas_call(
        _acc_body,
        grid=(NUM_BLOCKS,),
        in_specs=[
            pl.BlockSpec((ROWS_PER_BLOCK, NUM_COLS), lambda i: (i, 0)),
            pl.BlockSpec((1, 1, ROWS_PER_BLOCK), lambda i: (i, 0, 0)),
        ],
        out_specs=pl.BlockSpec((1, 1), lambda i: (0, 0)),
        out_shape=jax.ShapeDtypeStruct((1, 1), jnp.float32),
    )(output, lab3)
    return acc[0, 0]
